# baseline (speedup 1.0000x reference)
import jax
import jax.numpy as jnp
from jax import lax
from jax.experimental import pallas as pl
from jax.experimental.pallas import tpu as pltpu
from jax.experimental.pallas import tpu_sc as plsc

N = 20000
NP = 20480
NT = NP // 16
LANE = 16
CT = NT // LANE
UNROLL = 4
SB = CT // UNROLL
K = 100
KP = 128
KC = KP // LANE
SIGMA = 0.5
MIN_SCORE = 0.001
VOTING_THRESH = 0.75
NEG = -1e30
NEGINF = -3e38
BIGI = 2**30


def _nms_body(starts_h, ends_h, scores_h, cls_h,
              vox_h, voy_h, scout_h, clsout_h,
              s_v, e_v, so_v, wk_v, cl_v,
              pub_v, tmp_v, vote_v, vtmp_v,
              selsc_v, selcl_v, vx_v, vy_v, fsc_v,
              vox_st, voy_st, scout_st, clsout_st,
              shared_pub, shared_vote):
    c = lax.axis_index("c")
    s = lax.axis_index("s")
    base = s * NT
    lane = lax.broadcasted_iota(jnp.int32, (LANE,), 0)
    lane0 = lane == 0

    pltpu.sync_copy(starts_h.at[pl.ds(base, NT)], s_v)
    pltpu.sync_copy(ends_h.at[pl.ds(base, NT)], e_v)
    pltpu.sync_copy(scores_h.at[pl.ds(base, NT)], so_v)
    pltpu.sync_copy(scores_h.at[pl.ds(base, NT)], wk_v)
    pltpu.sync_copy(cls_h.at[pl.ds(base, NT)], cl_v)

    def amax_body(ci, bc):
        bv0, bi0 = bc
        v = wk_v[pl.ds(ci * LANE, LANE)]
        m = v > bv0
        return (jnp.where(m, v, bv0),
                jnp.where(m, ci * LANE + lane, bi0))
    bv_init = lax.fori_loop(
        0, CT, amax_body,
        (jnp.full((LANE,), NEGINF, jnp.float32),
         jnp.zeros((LANE,), jnp.int32)))

    def step(t, carry):
        bv, bi = carry
        lmax = jnp.max(bv)
        li = jnp.min(jnp.where(bv == lmax, bi, BIGI))
        liv = jnp.full((LANE,), li, jnp.int32)
        cst = plsc.load_gather(s_v, [liv])
        cen = plsc.load_gather(e_v, [liv])
        ccl = plsc.load_gather(cl_v, [liv])
        gi = li + base
        pv = jnp.where(
            lane == 0, plsc.bitcast(jnp.full((LANE,), lmax, jnp.float32),
                                    jnp.int32),
            jnp.where(lane == 1, jnp.full((LANE,), gi, jnp.int32),
            jnp.where(lane == 2, plsc.bitcast(cst, jnp.int32),
            jnp.where(lane == 3, plsc.bitcast(cen, jnp.int32),
            jnp.where(lane == 4, ccl, 0)))))
        pub_v[...] = pv
        pltpu.sync_copy(pub_v, shared_pub.at[s])
        plsc.subcore_barrier()
        pltpu.sync_copy(shared_pub, tmp_v)
        plsc.subcore_barrier()

        def col(j):
            return plsc.load_gather(
                tmp_v, [lane, jnp.full((LANE,), j, jnp.int32)])
        vals = plsc.bitcast(col(0), jnp.float32)
        gidxs = col(1)
        sts = plsc.bitcast(col(2), jnp.float32)
        ens = plsc.bitcast(col(3), jnp.float32)
        clss = col(4)
        GV = jnp.max(vals)
        mwin = vals == GV
        GI = jnp.min(jnp.where(mwin, gidxs, BIGI))
        m1 = mwin & (gidxs == GI)
        SS = jnp.sum(jnp.where(m1, sts, 0.0))
        SE = jnp.sum(jnp.where(m1, ens, 0.0))
        SCL = jnp.sum(jnp.where(m1, clss, 0))
        tv = jnp.full((LANE,), t, jnp.int32)
        plsc.store_scatter(selsc_v, [tv],
                           jnp.full((LANE,), GV, jnp.float32), mask=lane0)
        plsc.store_scatter(selcl_v, [tv],
                           jnp.full((LANE,), SCL, jnp.int32), mask=lane0)

        sel_len = SE - SS
        own = (GI >= base) & (GI < base + NT)
        jl = jnp.where(own, GI - base, -1)

        def dec_body(ci, acc):
            out = []
            for u in range(UNROLL):
                aw, ax, ay, nbv, nbi = acc[u]
                cc = u * SB + ci
                sv = s_v[pl.ds(cc * LANE, LANE)]
                ev = e_v[pl.ds(cc * LANE, LANE)]
                ov = so_v[pl.ds(cc * LANE, LANE)]
                wv = wk_v[pl.ds(cc * LANE, LANE)]
                left = jnp.maximum(SS, sv)
                right = jnp.minimum(SE, ev)
                inter = jnp.maximum(right - left, 0.0)
                uni = sel_len + (ev - sv) - inter
                iou = inter / jnp.maximum(uni, 1e-8)
                dec = jnp.exp(iou * iou * (-2.0))
                idxs = cc * LANE + lane
                nw = jnp.where(idxs == jl, NEG, wv * dec)
                wk_v[pl.ds(cc * LANE, LANE)] = nw
                w = jnp.where(iou >= VOTING_THRESH, ov * iou, 0.0)
                m = nw > nbv
                out.append((aw + w, ax + w * sv, ay + w * ev,
                            jnp.where(m, nw, nbv), jnp.where(m, idxs, nbi)))
            return tuple(out)
        z = jnp.zeros((LANE,), jnp.float32)
        acc0 = tuple(
            (z, z, z, jnp.full((LANE,), NEGINF, jnp.float32),
             jnp.zeros((LANE,), jnp.int32))
            for _ in range(UNROLL))
        accs = lax.fori_loop(0, SB, dec_body, acc0)
        aw, ax, ay, nbv, nbi = accs[0]
        for u in range(1, UNROLL):
            awu, axu, ayu, bvu, biu = accs[u]
            aw = aw + awu
            ax = ax + axu
            ay = ay + ayu
            gt = bvu > nbv
            eq = bvu == nbv
            nbi = jnp.where(gt, biu,
                            jnp.where(eq, jnp.minimum(nbi, biu), nbi))
            nbv = jnp.where(gt, bvu, nbv)

        z16 = jnp.zeros((LANE,), jnp.int32)
        plsc.store_scatter(vote_v, [z16, tv],
                           jnp.full((LANE,), jnp.sum(aw), jnp.float32),
                           mask=lane0)
        plsc.store_scatter(vote_v, [z16 + 1, tv],
                           jnp.full((LANE,), jnp.sum(ax), jnp.float32),
                           mask=lane0)
        plsc.store_scatter(vote_v, [z16 + 2, tv],
                           jnp.full((LANE,), jnp.sum(ay), jnp.float32),
                           mask=lane0)
        return nbv, nbi

    lax.fori_loop(0, K, step, bv_init)

    pltpu.sync_copy(vote_v, shared_vote.at[pl.ds(s * 3, 3)])
    plsc.subcore_barrier()
    writer = (c == 0) & (s == 0)

    @pl.when(writer)
    def _final():
        pltpu.sync_copy(shared_vote, vtmp_v)
        zf = jnp.zeros((LANE,), jnp.float32)
        zi = jnp.zeros((LANE,), jnp.int32)
        for cc in range(KC):
            accw = zf
            accx = zf
            accy = zf
            for r in range(16):
                accw = accw + vtmp_v[3 * r + 0, pl.ds(cc * LANE, LANE)]
                accx = accx + vtmp_v[3 * r + 1, pl.ds(cc * LANE, LANE)]
                accy = accy + vtmp_v[3 * r + 2, pl.ds(cc * LANE, LANE)]
            vx_v[pl.ds(cc * LANE, LANE)] = accx / accw
            vy_v[pl.ds(cc * LANE, LANE)] = accy / accw
            raw = selsc_v[pl.ds(cc * LANE, LANE)]
            filt = jnp.where(raw > MIN_SCORE, raw, 0.0)
            pos = cc * LANE + lane
            fsc_v[pl.ds(cc * LANE, LANE)] = jnp.where(pos < K, filt, NEGINF)
            vox_st[pl.ds(cc * LANE, LANE)] = zf
            voy_st[pl.ds(cc * LANE, LANE)] = zf
            scout_st[pl.ds(cc * LANE, LANE)] = zf
            clsout_st[pl.ds(cc * LANE, LANE)] = zi

        def sort_body(k, carry):
            bv = jnp.full((LANE,), NEGINF, jnp.float32)
            bt = jnp.zeros((LANE,), jnp.int32)
            for cc in range(KC):
                v = fsc_v[pl.ds(cc * LANE, LANE)]
                m = v > bv
                bv = jnp.where(m, v, bv)
                bt = jnp.where(m, cc * LANE + lane, bt)
            gm = jnp.max(bv)
            gt = jnp.min(jnp.where(bv == gm, bt, BIGI))
            gtv = jnp.full((LANE,), gt, jnp.int32)
            vx = plsc.load_gather(vx_v, [gtv])
            vy = plsc.load_gather(vy_v, [gtv])
            sv = plsc.load_gather(fsc_v, [gtv])
            cv = plsc.load_gather(selcl_v, [gtv])
            kv = jnp.full((LANE,), k, jnp.int32)
            plsc.store_scatter(vox_st, [kv], vx, mask=lane0)
            plsc.store_scatter(voy_st, [kv], vy, mask=lane0)
            plsc.store_scatter(scout_st, [kv], sv, mask=lane0)
            plsc.store_scatter(clsout_st, [kv], cv, mask=lane0)
            plsc.store_scatter(fsc_v, [gtv],
                               jnp.full((LANE,), NEGINF, jnp.float32),
                               mask=lane0)
            return carry
        lax.fori_loop(0, K, sort_body, 0)
        pltpu.sync_copy(vox_st, vox_h)
        pltpu.sync_copy(voy_st, voy_h)
        pltpu.sync_copy(scout_st, scout_h)
        pltpu.sync_copy(clsout_st, clsout_h)


_mesh = plsc.VectorSubcoreMesh(core_axis_name="c", subcore_axis_name="s")

_nms_call = pl.kernel(
    _nms_body,
    out_type=(
        jax.ShapeDtypeStruct((KP,), jnp.float32),
        jax.ShapeDtypeStruct((KP,), jnp.float32),
        jax.ShapeDtypeStruct((KP,), jnp.float32),
        jax.ShapeDtypeStruct((KP,), jnp.int32),
    ),
    mesh=_mesh,
    compiler_params=pltpu.CompilerParams(needs_layout_passes=False),
    scratch_types=[
        pltpu.VMEM((NT,), jnp.float32),
        pltpu.VMEM((NT,), jnp.float32),
        pltpu.VMEM((NT,), jnp.float32),
        pltpu.VMEM((NT,), jnp.float32),
        pltpu.VMEM((NT,), jnp.int32),
        pltpu.VMEM((LANE,), jnp.int32),
        pltpu.VMEM((16, LANE), jnp.int32),
        pltpu.VMEM((3, KP), jnp.float32),
        pltpu.VMEM((48, KP), jnp.float32),
        pltpu.VMEM((KP,), jnp.float32),
        pltpu.VMEM((KP,), jnp.int32),
        pltpu.VMEM((KP,), jnp.float32),
        pltpu.VMEM((KP,), jnp.float32),
        pltpu.VMEM((KP,), jnp.float32),
        pltpu.VMEM((KP,), jnp.float32),
        pltpu.VMEM((KP,), jnp.float32),
        pltpu.VMEM((KP,), jnp.float32),
        pltpu.VMEM((KP,), jnp.int32),
        pltpu.VMEM_SHARED((16, LANE), jnp.int32),
        pltpu.VMEM_SHARED((48, KP), jnp.float32),
    ],
)


def kernel(segs, scores, cls_idxs):
    pad = NP - N
    starts = jnp.concatenate(
        [segs[:, 0], jnp.full((pad,), -1e6 - 128.0, jnp.float32)])
    ends = jnp.concatenate(
        [segs[:, 1], jnp.full((pad,), -1e6, jnp.float32)])
    sc = jnp.concatenate([scores, jnp.full((pad,), NEG, jnp.float32)])
    cl = jnp.concatenate([cls_idxs, jnp.zeros((pad,), jnp.int32)])
    vox, voy, scout, clsout = _nms_call(starts, ends, sc, cl)
    voted = jnp.stack([vox[:K], voy[:K]], axis=1)
    return voted, scout[:K], clsout[:K]


# speedup vs baseline: 2.1209x; 2.1209x over previous
import jax
import jax.numpy as jnp
from jax import lax
from jax.experimental import pallas as pl
from jax.experimental.pallas import tpu as pltpu
from jax.experimental.pallas import tpu_sc as plsc

N = 20000
NP = 20480
NT = NP // 16
LANE = 16
CT = NT // LANE
UNROLL = 4
SB = CT // UNROLL
K = 100
KP = 128
KC = KP // LANE
SIGMA = 0.5
MIN_SCORE = 0.001
VOTING_THRESH = 0.75
NEG = -1e30
NEGINF = -3e38
BIGI = 2**30


def _nms_body(starts_h, ends_h, scores_h, cls_h,
              vox_h, voy_h, scout_h, clsout_h,
              s_v, e_v, so_v, wk0, wk1, wk2, wk3, cl_v,
              pub_v, tmp_v, vote_v, vtmp_v,
              selsc_v, selcl_v, vx_v, vy_v, fsc_v,
              vox_st, voy_st, scout_st, clsout_st,
              shared_pub, shared_vote):
    c = lax.axis_index("c")
    s = lax.axis_index("s")
    base = s * NT
    lane = lax.broadcasted_iota(jnp.int32, (LANE,), 0)
    lane0 = lane == 0

    pltpu.sync_copy(starts_h.at[pl.ds(base, NT)], s_v)
    pltpu.sync_copy(ends_h.at[pl.ds(base, NT)], e_v)
    pltpu.sync_copy(scores_h.at[pl.ds(base, NT)], so_v)
    wks = (wk0, wk1, wk2, wk3)
    NSB = SB * LANE
    for u in range(UNROLL):
        pltpu.sync_copy(scores_h.at[pl.ds(base + u * NSB, NSB)], wks[u])
    pltpu.sync_copy(cls_h.at[pl.ds(base, NT)], cl_v)

    def amax_body(ci, bc):
        out = []
        for u in range(UNROLL):
            bv0, bi0 = bc[u]
            v = wks[u][pl.ds(ci * LANE, LANE)]
            m = v > bv0
            out.append((jnp.where(m, v, bv0),
                        jnp.where(m, u * NSB + ci * LANE + lane, bi0)))
        return tuple(out)
    acc_init = lax.fori_loop(
        0, SB, amax_body,
        tuple((jnp.full((LANE,), NEGINF, jnp.float32),
               jnp.zeros((LANE,), jnp.int32)) for _ in range(UNROLL)))
    bvm, bim = acc_init[0]
    for u in range(1, UNROLL):
        bvu, biu = acc_init[u]
        gt0 = bvu > bvm
        eq0 = bvu == bvm
        bim = jnp.where(gt0, biu,
                        jnp.where(eq0, jnp.minimum(bim, biu), bim))
        bvm = jnp.where(gt0, bvu, bvm)
    bv_init = (bvm, bim)

    def step(t, carry):
        bv, bi = carry
        lmax = jnp.max(bv)
        li = jnp.min(jnp.where(bv == lmax, bi, BIGI))
        liv = jnp.full((LANE,), li, jnp.int32)
        cst = plsc.load_gather(s_v, [liv])
        cen = plsc.load_gather(e_v, [liv])
        ccl = plsc.load_gather(cl_v, [liv])
        gi = li + base
        pv = jnp.where(
            lane == 0, plsc.bitcast(jnp.full((LANE,), lmax, jnp.float32),
                                    jnp.int32),
            jnp.where(lane == 1, jnp.full((LANE,), gi, jnp.int32),
            jnp.where(lane == 2, plsc.bitcast(cst, jnp.int32),
            jnp.where(lane == 3, plsc.bitcast(cen, jnp.int32),
            jnp.where(lane == 4, ccl, 0)))))
        pub_v[...] = pv
        pltpu.sync_copy(pub_v, shared_pub.at[s])
        plsc.subcore_barrier()
        pltpu.sync_copy(shared_pub, tmp_v)
        plsc.subcore_barrier()

        def col(j):
            return plsc.load_gather(
                tmp_v, [lane, jnp.full((LANE,), j, jnp.int32)])
        vals = plsc.bitcast(col(0), jnp.float32)
        gidxs = col(1)
        sts = plsc.bitcast(col(2), jnp.float32)
        ens = plsc.bitcast(col(3), jnp.float32)
        clss = col(4)
        GV = jnp.max(vals)
        mwin = vals == GV
        GI = jnp.min(jnp.where(mwin, gidxs, BIGI))
        m1 = mwin & (gidxs == GI)
        SS = jnp.sum(jnp.where(m1, sts, 0.0))
        SE = jnp.sum(jnp.where(m1, ens, 0.0))
        SCL = jnp.sum(jnp.where(m1, clss, 0))
        tv = jnp.full((LANE,), t, jnp.int32)
        plsc.store_scatter(selsc_v, [tv],
                           jnp.full((LANE,), GV, jnp.float32), mask=lane0)
        plsc.store_scatter(selcl_v, [tv],
                           jnp.full((LANE,), SCL, jnp.int32), mask=lane0)

        sel_len = SE - SS
        own = (GI >= base) & (GI < base + NT)
        jl = jnp.where(own, GI - base, -1)

        def dec_body(ci, acc):
            out = []
            for u in range(UNROLL):
                aw, ax, ay, nbv, nbi = acc[u]
                cc = u * SB + ci
                sv = s_v[pl.ds(cc * LANE, LANE)]
                ev = e_v[pl.ds(cc * LANE, LANE)]
                ov = so_v[pl.ds(cc * LANE, LANE)]
                wv = wks[u][pl.ds(ci * LANE, LANE)]
                left = jnp.maximum(SS, sv)
                right = jnp.minimum(SE, ev)
                inter = jnp.maximum(right - left, 0.0)
                uni = sel_len + (ev - sv) - inter
                iou = inter / jnp.maximum(uni, 1e-8)
                dec = jnp.exp(iou * iou * (-2.0))
                idxs = cc * LANE + lane
                nw = jnp.where(idxs == jl, NEG, wv * dec)
                wks[u][pl.ds(ci * LANE, LANE)] = nw
                w = jnp.where(iou >= VOTING_THRESH, ov * iou, 0.0)
                m = nw > nbv
                out.append((aw + w, ax + w * sv, ay + w * ev,
                            jnp.where(m, nw, nbv), jnp.where(m, idxs, nbi)))
            return tuple(out)
        z = jnp.zeros((LANE,), jnp.float32)
        acc0 = tuple(
            (z, z, z, jnp.full((LANE,), NEGINF, jnp.float32),
             jnp.zeros((LANE,), jnp.int32))
            for _ in range(UNROLL))
        accs = lax.fori_loop(0, SB, dec_body, acc0)
        aw, ax, ay, nbv, nbi = accs[0]
        for u in range(1, UNROLL):
            awu, axu, ayu, bvu, biu = accs[u]
            aw = aw + awu
            ax = ax + axu
            ay = ay + ayu
            gt = bvu > nbv
            eq = bvu == nbv
            nbi = jnp.where(gt, biu,
                            jnp.where(eq, jnp.minimum(nbi, biu), nbi))
            nbv = jnp.where(gt, bvu, nbv)

        z16 = jnp.zeros((LANE,), jnp.int32)
        plsc.store_scatter(vote_v, [z16, tv],
                           jnp.full((LANE,), jnp.sum(aw), jnp.float32),
                           mask=lane0)
        plsc.store_scatter(vote_v, [z16 + 1, tv],
                           jnp.full((LANE,), jnp.sum(ax), jnp.float32),
                           mask=lane0)
        plsc.store_scatter(vote_v, [z16 + 2, tv],
                           jnp.full((LANE,), jnp.sum(ay), jnp.float32),
                           mask=lane0)
        return nbv, nbi

    lax.fori_loop(0, K, step, bv_init)

    pltpu.sync_copy(vote_v, shared_vote.at[pl.ds(s * 3, 3)])
    plsc.subcore_barrier()
    writer = (c == 0) & (s == 0)

    @pl.when(writer)
    def _final():
        pltpu.sync_copy(shared_vote, vtmp_v)
        zf = jnp.zeros((LANE,), jnp.float32)
        zi = jnp.zeros((LANE,), jnp.int32)
        for cc in range(KC):
            accw = zf
            accx = zf
            accy = zf
            for r in range(16):
                accw = accw + vtmp_v[3 * r + 0, pl.ds(cc * LANE, LANE)]
                accx = accx + vtmp_v[3 * r + 1, pl.ds(cc * LANE, LANE)]
                accy = accy + vtmp_v[3 * r + 2, pl.ds(cc * LANE, LANE)]
            vx_v[pl.ds(cc * LANE, LANE)] = accx / accw
            vy_v[pl.ds(cc * LANE, LANE)] = accy / accw
            raw = selsc_v[pl.ds(cc * LANE, LANE)]
            filt = jnp.where(raw > MIN_SCORE, raw, 0.0)
            pos = cc * LANE + lane
            fsc_v[pl.ds(cc * LANE, LANE)] = jnp.where(pos < K, filt, NEGINF)
            vox_st[pl.ds(cc * LANE, LANE)] = zf
            voy_st[pl.ds(cc * LANE, LANE)] = zf
            scout_st[pl.ds(cc * LANE, LANE)] = zf
            clsout_st[pl.ds(cc * LANE, LANE)] = zi

        def sort_body(k, carry):
            bv = jnp.full((LANE,), NEGINF, jnp.float32)
            bt = jnp.zeros((LANE,), jnp.int32)
            for cc in range(KC):
                v = fsc_v[pl.ds(cc * LANE, LANE)]
                m = v > bv
                bv = jnp.where(m, v, bv)
                bt = jnp.where(m, cc * LANE + lane, bt)
            gm = jnp.max(bv)
            gt = jnp.min(jnp.where(bv == gm, bt, BIGI))
            gtv = jnp.full((LANE,), gt, jnp.int32)
            vx = plsc.load_gather(vx_v, [gtv])
            vy = plsc.load_gather(vy_v, [gtv])
            sv = plsc.load_gather(fsc_v, [gtv])
            cv = plsc.load_gather(selcl_v, [gtv])
            kv = jnp.full((LANE,), k, jnp.int32)
            plsc.store_scatter(vox_st, [kv], vx, mask=lane0)
            plsc.store_scatter(voy_st, [kv], vy, mask=lane0)
            plsc.store_scatter(scout_st, [kv], sv, mask=lane0)
            plsc.store_scatter(clsout_st, [kv], cv, mask=lane0)
            plsc.store_scatter(fsc_v, [gtv],
                               jnp.full((LANE,), NEGINF, jnp.float32),
                               mask=lane0)
            return carry
        lax.fori_loop(0, K, sort_body, 0)
        pltpu.sync_copy(vox_st, vox_h)
        pltpu.sync_copy(voy_st, voy_h)
        pltpu.sync_copy(scout_st, scout_h)
        pltpu.sync_copy(clsout_st, clsout_h)


_mesh = plsc.VectorSubcoreMesh(core_axis_name="c", subcore_axis_name="s")

_nms_call = pl.kernel(
    _nms_body,
    out_type=(
        jax.ShapeDtypeStruct((KP,), jnp.float32),
        jax.ShapeDtypeStruct((KP,), jnp.float32),
        jax.ShapeDtypeStruct((KP,), jnp.float32),
        jax.ShapeDtypeStruct((KP,), jnp.int32),
    ),
    mesh=_mesh,
    compiler_params=pltpu.CompilerParams(needs_layout_passes=False),
    scratch_types=[
        pltpu.VMEM((NT,), jnp.float32),
        pltpu.VMEM((NT,), jnp.float32),
        pltpu.VMEM((NT,), jnp.float32),
        pltpu.VMEM((NT // UNROLL,), jnp.float32),
        pltpu.VMEM((NT // UNROLL,), jnp.float32),
        pltpu.VMEM((NT // UNROLL,), jnp.float32),
        pltpu.VMEM((NT // UNROLL,), jnp.float32),
        pltpu.VMEM((NT,), jnp.int32),
        pltpu.VMEM((LANE,), jnp.int32),
        pltpu.VMEM((16, LANE), jnp.int32),
        pltpu.VMEM((3, KP), jnp.float32),
        pltpu.VMEM((48, KP), jnp.float32),
        pltpu.VMEM((KP,), jnp.float32),
        pltpu.VMEM((KP,), jnp.int32),
        pltpu.VMEM((KP,), jnp.float32),
        pltpu.VMEM((KP,), jnp.float32),
        pltpu.VMEM((KP,), jnp.float32),
        pltpu.VMEM((KP,), jnp.float32),
        pltpu.VMEM((KP,), jnp.float32),
        pltpu.VMEM((KP,), jnp.float32),
        pltpu.VMEM((KP,), jnp.int32),
        pltpu.VMEM_SHARED((16, LANE), jnp.int32),
        pltpu.VMEM_SHARED((48, KP), jnp.float32),
    ],
)


def kernel(segs, scores, cls_idxs):
    pad = NP - N
    starts = jnp.concatenate(
        [segs[:, 0], jnp.full((pad,), -1e6 - 128.0, jnp.float32)])
    ends = jnp.concatenate(
        [segs[:, 1], jnp.full((pad,), -1e6, jnp.float32)])
    sc = jnp.concatenate([scores, jnp.full((pad,), NEG, jnp.float32)])
    cl = jnp.concatenate([cls_idxs, jnp.zeros((pad,), jnp.int32)])
    vox, voy, scout, clsout = _nms_call(starts, ends, sc, cl)
    voted = jnp.stack([vox[:K], voy[:K]], axis=1)
    return voted, scout[:K], clsout[:K]
